# baseline (speedup 1.0000x reference)
import functools

import jax
import jax.numpy as jnp
from jax.experimental import pallas as pl
from jax.experimental.pallas import tpu as pltpu

D = 2048
N_TOK = 32768
B = 16
R = 1024
CW = 256
NEG = -1e30


def _pool_body(idx_ref, watt_ref, batt_ref, a_ref, acc_ref, l_ref, m_ref):
    i = pl.program_id(0)

    @pl.when(i == 0)
    def _init():
        m_ref[...] = jnp.full_like(m_ref, NEG)
        l_ref[...] = jnp.zeros_like(l_ref)
        acc_ref[...] = jnp.zeros_like(acc_ref)

    a = a_ref[...]
    s = jax.lax.dot_general(
        a, watt_ref[...], (((1,), (0,)), ((), ())),
        preferred_element_type=jnp.float32) + batt_ref[0, 0]

    pos = i * R + jax.lax.broadcasted_iota(jnp.int32, (R, B), 0)
    st = idx_ref[...][:, 0][None, :]
    en = idx_ref[...][:, 1][None, :]
    mask = (pos >= st) & (pos <= en)

    sb = jnp.where(mask, s, NEG)
    bm = jnp.max(sb, axis=0)
    m_old = m_ref[0]
    m_new = jnp.maximum(m_old, bm)
    alpha = jnp.exp(m_old - m_new)
    e = jnp.exp(sb - m_new[None, :])
    l_ref[0] = alpha * l_ref[0] + jnp.sum(e, axis=0)
    m_ref[0] = m_new
    acc_ref[...] = acc_ref[...] * alpha[:, None] + jax.lax.dot_general(
        e, a, (((0,), (0,)), ((), ())),
        preferred_element_type=jnp.float32)


def _proj_body(acc_ref, l_ref, wout_ref, bout_ref, out_ref):
    pooled = acc_ref[...] / l_ref[0][:, None]
    out_ref[...] = jax.lax.dot_general(
        pooled, wout_ref[...], (((1,), (0,)), ((), ())),
        preferred_element_type=jnp.float32) + bout_ref[...]


@jax.jit
def kernel(atom_features, index_list, W_att, b_att, W_out, b_out):
    nb = N_TOK // R
    acc, l = pl.pallas_call(
        _pool_body,
        grid=(nb,),
        in_specs=[
            pl.BlockSpec((B, 2), lambda i: (0, 0)),
            pl.BlockSpec((D, 1), lambda i: (0, 0)),
            pl.BlockSpec((1, 1), lambda i: (0, 0)),
            pl.BlockSpec((R, D), lambda i: (i, 0)),
        ],
        out_specs=[
            pl.BlockSpec((B, D), lambda i: (0, 0)),
            pl.BlockSpec((1, B), lambda i: (0, 0)),
        ],
        out_shape=[
            jax.ShapeDtypeStruct((B, D), jnp.float32),
            jax.ShapeDtypeStruct((1, B), jnp.float32),
        ],
        scratch_shapes=[
            pltpu.VMEM((1, B), jnp.float32),
        ],
    )(index_list.astype(jnp.int32), W_att, b_att.reshape(1, 1),
      atom_features)

    return pl.pallas_call(
        _proj_body,
        grid=(D // CW,),
        in_specs=[
            pl.BlockSpec((B, D), lambda j: (0, 0)),
            pl.BlockSpec((1, B), lambda j: (0, 0)),
            pl.BlockSpec((D, CW), lambda j: (0, j)),
            pl.BlockSpec((1, CW), lambda j: (0, j)),
        ],
        out_specs=pl.BlockSpec((B, CW), lambda j: (0, j)),
        out_shape=jax.ShapeDtypeStruct((B, D), jnp.float32),
    )(acc, l, W_out, b_out.reshape(1, D))


# speedup vs baseline: 1.2097x; 1.2097x over previous
import jax
import jax.numpy as jnp
from jax.experimental import pallas as pl
from jax.experimental.pallas import tpu as pltpu

D = 2048
N_TOK = 32768
B = 16
R = 2048
H = D // 2
CW = 256
NEG = -1e30


def _pool_body(idx_ref, watt_ref, batt_ref, a0_ref, a1_ref,
               acc0_ref, acc1_ref, l_ref, m_ref):
    i = pl.program_id(0)

    @pl.when(i == 0)
    def _init():
        m_ref[...] = jnp.full_like(m_ref, NEG)
        l_ref[...] = jnp.zeros_like(l_ref)
        acc0_ref[...] = jnp.zeros_like(acc0_ref)
        acc1_ref[...] = jnp.zeros_like(acc1_ref)

    a0 = a0_ref[...]
    a1 = a1_ref[...]
    w = watt_ref[...]
    s = (jax.lax.dot_general(a0, w[:H], (((1,), (0,)), ((), ())),
                             preferred_element_type=jnp.float32)
         + jax.lax.dot_general(a1, w[H:], (((1,), (0,)), ((), ())),
                               preferred_element_type=jnp.float32)
         + batt_ref[0, 0])

    pos = i * R + jax.lax.broadcasted_iota(jnp.int32, (R, B), 0)
    st = idx_ref[...][:, 0][None, :]
    en = idx_ref[...][:, 1][None, :]
    mask = (pos >= st) & (pos <= en)

    sb = jnp.where(mask, s, NEG)
    bm = jnp.max(sb, axis=0)
    m_old = m_ref[0]
    m_new = jnp.maximum(m_old, bm)
    alpha = jnp.exp(m_old - m_new)
    e = jnp.exp(sb - m_new[None, :])
    l_ref[0] = alpha * l_ref[0] + jnp.sum(e, axis=0)
    m_ref[0] = m_new
    acc0_ref[...] = acc0_ref[...] * alpha[:, None] + jax.lax.dot_general(
        e, a0, (((0,), (0,)), ((), ())),
        preferred_element_type=jnp.float32)
    acc1_ref[...] = acc1_ref[...] * alpha[:, None] + jax.lax.dot_general(
        e, a1, (((0,), (0,)), ((), ())),
        preferred_element_type=jnp.float32)


def _proj_body(acc0_ref, acc1_ref, l_ref, w0_ref, w1_ref, bout_ref, out_ref):
    p0 = acc0_ref[...] / l_ref[0][:, None]
    p1 = acc1_ref[...] / l_ref[0][:, None]
    out_ref[...] = (
        jax.lax.dot_general(p0, w0_ref[...], (((1,), (0,)), ((), ())),
                            preferred_element_type=jnp.float32)
        + jax.lax.dot_general(p1, w1_ref[...], (((1,), (0,)), ((), ())),
                              preferred_element_type=jnp.float32)
        + bout_ref[...])


@jax.jit
def kernel(atom_features, index_list, W_att, b_att, W_out, b_out):
    nb = N_TOK // R
    acc0, acc1, l = pl.pallas_call(
        _pool_body,
        grid=(nb,),
        in_specs=[
            pl.BlockSpec((B, 2), lambda i: (0, 0)),
            pl.BlockSpec((D, 1), lambda i: (0, 0)),
            pl.BlockSpec((1, 1), lambda i: (0, 0)),
            pl.BlockSpec((R, H), lambda i: (i, 0)),
            pl.BlockSpec((R, H), lambda i: (i, 1)),
        ],
        out_specs=[
            pl.BlockSpec((B, H), lambda i: (0, 0)),
            pl.BlockSpec((B, H), lambda i: (0, 0)),
            pl.BlockSpec((1, B), lambda i: (0, 0)),
        ],
        out_shape=[
            jax.ShapeDtypeStruct((B, H), jnp.float32),
            jax.ShapeDtypeStruct((B, H), jnp.float32),
            jax.ShapeDtypeStruct((1, B), jnp.float32),
        ],
        scratch_shapes=[
            pltpu.VMEM((1, B), jnp.float32),
        ],
    )(index_list.astype(jnp.int32), W_att, b_att.reshape(1, 1),
      atom_features, atom_features)

    return pl.pallas_call(
        _proj_body,
        grid=(D // CW,),
        in_specs=[
            pl.BlockSpec((B, H), lambda j: (0, 0)),
            pl.BlockSpec((B, H), lambda j: (0, 0)),
            pl.BlockSpec((1, B), lambda j: (0, 0)),
            pl.BlockSpec((H, CW), lambda j: (0, j)),
            pl.BlockSpec((H, CW), lambda j: (1, j)),
            pl.BlockSpec((1, CW), lambda j: (0, j)),
        ],
        out_specs=pl.BlockSpec((B, CW), lambda j: (0, j)),
        out_shape=jax.ShapeDtypeStruct((B, D), jnp.float32),
    )(acc0, acc1, l, W_out, W_out, b_out.reshape(1, D))
